# baseline (speedup 1.0000x reference)
import functools

import jax
import jax.numpy as jnp
from jax import lax
from jax.experimental import pallas as pl
from jax.experimental.pallas import tpu as pltpu
from jax.experimental.pallas import tpu_sc as plsc

N = 10000
E = 320000
D = 128
NL = 3

NC = 2
NS = 16
NW = NC * NS

CHUNK = 128
N_CHUNKS = 80
HALF = N_CHUNKS // 2
E_TILE = N_CHUNKS * CHUNK
EP = NW * E_TILE
N_ACC = N + 8
STRIPE = 624
TAIL0 = NS * STRIPE
TAIL = N - TAIL0


def _sc_partial_segsum(h, src_r, dst_r, z):
  mesh = plsc.VectorSubcoreMesh(core_axis_name="c", subcore_axis_name="s")

  @functools.partial(
      pl.kernel,
      out_type=jax.ShapeDtypeStruct((NC, N, D), jnp.float32),
      mesh=mesh,
      scratch_types=[
          pltpu.VMEM((N_CHUNKS, CHUNK), jnp.int32),
          pltpu.VMEM((HALF, CHUNK), jnp.int32),
          pltpu.VMEM((CHUNK, D), jnp.float32),
          pltpu.VMEM((CHUNK, D), jnp.float32),
          pltpu.VMEM_SHARED((N_ACC, D), jnp.float32),
          pltpu.SemaphoreType.DMA,
          pltpu.SemaphoreType.DMA,
      ],
  )
  def k(h_hbm, src_hbm, dst_hbm, z_hbm, out_hbm, src_v, dst_v,
        rows0, rows1, acc_sh, sem0, sem1):
    cid = lax.axis_index("c")
    sid = lax.axis_index("s")
    wid = cid * NS + sid
    row0 = sid * STRIPE
    pltpu.sync_copy(z_hbm.at[pl.ds(row0, STRIPE)],
                    acc_sh.at[pl.ds(row0, STRIPE)])

    @pl.when(sid == NS - 1)
    def _():
      pltpu.sync_copy(z_hbm.at[pl.ds(TAIL0, TAIL)], acc_sh.at[pl.ds(TAIL0, TAIL)])

    pltpu.sync_copy(src_hbm.at[wid], src_v)
    plsc.subcore_barrier()

    def scatter(buf, c):
      pltpu.sync_copy(buf, acc_sh.at[dst_v.at[c]], add=True)

    for half in range(2):
      pltpu.sync_copy(dst_hbm.at[wid].at[half], dst_v)
      base = half * HALF

      @pl.loop(0, HALF)
      def _(c):
        pltpu.async_copy(h_hbm.at[src_v.at[base + c]], rows0, sem0).wait()
        scatter(rows0, c)

    plsc.subcore_barrier()
    pltpu.sync_copy(acc_sh.at[pl.ds(row0, STRIPE)],
                    out_hbm.at[cid].at[pl.ds(row0, STRIPE)])

    @pl.when(sid == NS - 1)
    def _():
      pltpu.sync_copy(acc_sh.at[pl.ds(TAIL0, TAIL)],
                      out_hbm.at[cid].at[pl.ds(TAIL0, TAIL)])

  return k(h, src_r, dst_r, z)


def _tc_update(h, A, Wm, Ws, Wu, bias):
  BLK = 1000

  def body(h_ref, a0_ref, a1_ref, wm_ref, ws_ref, wu_ref, b_ref, o_ref):
    a = a0_ref[...] + a1_ref[...]
    agg = jnp.dot(a, wm_ref[...], preferred_element_type=jnp.float32)
    out = (jnp.dot(h_ref[...], ws_ref[...], preferred_element_type=jnp.float32)
           + jnp.dot(agg, wu_ref[...], preferred_element_type=jnp.float32)
           + b_ref[...])
    o_ref[...] = jnp.maximum(out, 0.0)

  return pl.pallas_call(
      body,
      grid=(N // BLK,),
      in_specs=[
          pl.BlockSpec((BLK, D), lambda i: (i, 0)),
          pl.BlockSpec((BLK, D), lambda i: (i, 0)),
          pl.BlockSpec((BLK, D), lambda i: (i, 0)),
          pl.BlockSpec((D, D), lambda i: (0, 0)),
          pl.BlockSpec((D, D), lambda i: (0, 0)),
          pl.BlockSpec((D, D), lambda i: (0, 0)),
          pl.BlockSpec((1, D), lambda i: (0, 0)),
      ],
      out_specs=pl.BlockSpec((BLK, D), lambda i: (i, 0)),
      out_shape=jax.ShapeDtypeStruct((N, D), jnp.float32),
  )(h, A[0], A[1], Wm, Ws, Wu, bias)


def kernel(x, edge_index, W_msg, W_self, W_upd, b):
  pad = EP - E
  src = jnp.concatenate(
      [edge_index[0].astype(jnp.int32), jnp.zeros((pad,), jnp.int32)])
  dst = jnp.concatenate(
      [edge_index[1].astype(jnp.int32), jnp.full((pad,), N, jnp.int32)])
  src = src.reshape(NW, N_CHUNKS, CHUNK)
  dst = dst.reshape(NW, 2, HALF, CHUNK)
  z = jnp.zeros((N, D), jnp.float32)
  bias = b.reshape(NL, 1, D)
  h = x
  for l in range(NL):
    A = _sc_partial_segsum(h, src, dst, z)
    h = _tc_update(h, A, W_msg[l], W_self[l], W_upd[l], bias[l])
  return h


# speedup vs baseline: 4.0162x; 4.0162x over previous
import functools

import jax
import jax.numpy as jnp
from jax import lax
from jax.experimental import pallas as pl
from jax.experimental.pallas import tpu as pltpu
from jax.experimental.pallas import tpu_sc as plsc

N = 10000
E = 320000
D = 128
NL = 3

NC = 2
NS = 16
NW = NC * NS

E_TILE = E // NW
CHUNK = 128
N_CHUNKS = 78
HALF = N_CHUNKS // 2
XTRA = E_TILE - N_CHUNKS * CHUNK
STRIPE = 624
TAIL0 = NS * STRIPE
TAIL = N - TAIL0


def _sc_partial_segsum(h, src_r, dst_r, srcx, dstx, z):
  mesh = plsc.VectorSubcoreMesh(core_axis_name="c", subcore_axis_name="s")

  @functools.partial(
      pl.kernel,
      out_type=jax.ShapeDtypeStruct((NC, N, D), jnp.float32),
      mesh=mesh,
      scratch_types=[
          pltpu.VMEM((N_CHUNKS, CHUNK), jnp.int32),
          pltpu.VMEM((HALF, CHUNK), jnp.int32),
          pltpu.VMEM((XTRA,), jnp.int32),
          pltpu.VMEM((XTRA,), jnp.int32),
          pltpu.VMEM((CHUNK, D), jnp.float32),
          pltpu.VMEM((CHUNK, D), jnp.float32),
          pltpu.VMEM_SHARED((N, D), jnp.float32),
          pltpu.SemaphoreType.DMA,
          pltpu.SemaphoreType.DMA,
      ],
  )
  def k(h_hbm, src_hbm, dst_hbm, srcx_hbm, dstx_hbm, z_hbm, out_hbm,
        src_v, dst_v, srcx_v, dstx_v, rows0, rows1, acc_sh, sem0, sem1):
    cid = lax.axis_index("c")
    sid = lax.axis_index("s")
    wid = cid * NS + sid
    row0 = sid * STRIPE
    pltpu.sync_copy(z_hbm.at[pl.ds(row0, STRIPE)],
                    acc_sh.at[pl.ds(row0, STRIPE)])

    @pl.when(sid == NS - 1)
    def _():
      pltpu.sync_copy(z_hbm.at[pl.ds(TAIL0, TAIL)], acc_sh.at[pl.ds(TAIL0, TAIL)])

    pltpu.sync_copy(src_hbm.at[wid], src_v)
    pltpu.sync_copy(srcx_hbm.at[wid], srcx_v)
    pltpu.sync_copy(dstx_hbm.at[wid], dstx_v)
    plsc.subcore_barrier()

    bufs = ((rows0, sem0), (rows1, sem1))

    def fire(chunk, buf):
      pltpu.async_copy(h_hbm.at[src_v.at[chunk]], buf[0], buf[1])

    def wait(buf):
      pltpu.make_async_copy(h_hbm.at[pl.ds(0, CHUNK)], buf[0], buf[1]).wait()

    def scatter(buf, c):
      pltpu.sync_copy(buf[0], acc_sh.at[dst_v.at[c]], add=True)

    fire(0, bufs[0])
    for half in range(2):
      pltpu.sync_copy(dst_hbm.at[wid].at[half], dst_v)
      base = half * HALF
      lead, other = bufs[half], bufs[1 - half]

      @pl.loop(0, HALF - 2, step=2)
      def _(c):
        fire(base + c + 1, other)
        wait(lead)
        scatter(lead, c)
        fire(base + c + 2, lead)
        wait(other)
        scatter(other, c + 1)

      if half == 0:
        fire(HALF, other)
      wait(lead)
      scatter(lead, HALF - 1)

    pltpu.async_copy(h_hbm.at[srcx_v], rows0.at[pl.ds(0, XTRA)], sem0).wait()
    pltpu.sync_copy(rows0.at[pl.ds(0, XTRA)], acc_sh.at[dstx_v], add=True)

    plsc.subcore_barrier()
    pltpu.sync_copy(acc_sh.at[pl.ds(row0, STRIPE)],
                    out_hbm.at[cid].at[pl.ds(row0, STRIPE)])

    @pl.when(sid == NS - 1)
    def _():
      pltpu.sync_copy(acc_sh.at[pl.ds(TAIL0, TAIL)],
                      out_hbm.at[cid].at[pl.ds(TAIL0, TAIL)])

  return k(h, src_r, dst_r, srcx, dstx, z)


def _tc_update(h, A, Wm, Ws, Wu, bias):
  BLK = 1000

  def body(h_ref, a0_ref, a1_ref, wm_ref, ws_ref, wu_ref, b_ref, o_ref):
    a = a0_ref[...] + a1_ref[...]
    agg = jnp.dot(a, wm_ref[...], preferred_element_type=jnp.float32)
    out = (jnp.dot(h_ref[...], ws_ref[...], preferred_element_type=jnp.float32)
           + jnp.dot(agg, wu_ref[...], preferred_element_type=jnp.float32)
           + b_ref[...])
    o_ref[...] = jnp.maximum(out, 0.0)

  return pl.pallas_call(
      body,
      grid=(N // BLK,),
      in_specs=[
          pl.BlockSpec((BLK, D), lambda i: (i, 0)),
          pl.BlockSpec((BLK, D), lambda i: (i, 0)),
          pl.BlockSpec((BLK, D), lambda i: (i, 0)),
          pl.BlockSpec((D, D), lambda i: (0, 0)),
          pl.BlockSpec((D, D), lambda i: (0, 0)),
          pl.BlockSpec((D, D), lambda i: (0, 0)),
          pl.BlockSpec((1, D), lambda i: (0, 0)),
      ],
      out_specs=pl.BlockSpec((BLK, D), lambda i: (i, 0)),
      out_shape=jax.ShapeDtypeStruct((N, D), jnp.float32),
  )(h, A[0], A[1], Wm, Ws, Wu, bias)


def kernel(x, edge_index, W_msg, W_self, W_upd, b):
  ei = edge_index.astype(jnp.int32).reshape(2, NW, E_TILE)
  main = N_CHUNKS * CHUNK
  src = ei[0, :, :main].reshape(NW, N_CHUNKS, CHUNK)
  dst = ei[1, :, :main].reshape(NW, 2, HALF, CHUNK)
  srcx = ei[0, :, main:]
  dstx = ei[1, :, main:]
  z = jnp.zeros((N, D), jnp.float32)
  bias = b.reshape(NL, 1, D)
  h = x
  for l in range(NL):
    A = _sc_partial_segsum(h, src, dst, srcx, dstx, z)
    h = _tc_update(h, A, W_msg[l], W_self[l], W_upd[l], bias[l])
  return h


# speedup vs baseline: 4.0274x; 1.0028x over previous
import functools

import jax
import jax.numpy as jnp
from jax import lax
from jax.experimental import pallas as pl
from jax.experimental.pallas import tpu as pltpu
from jax.experimental.pallas import tpu_sc as plsc

N = 10000
E = 320000
D = 128
NL = 3

NC = 2
NS = 16
NW = NC * NS

E_TILE = E // NW
CHUNK = 128
N_CHUNKS = 78
HALF = N_CHUNKS // 2
XTRA = E_TILE - N_CHUNKS * CHUNK
STRIPE = 624
TAIL0 = NS * STRIPE
TAIL = N - TAIL0


def _sc_partial_segsum(h, src_r, dst_r, srcx, dstx, z):
  mesh = plsc.VectorSubcoreMesh(core_axis_name="c", subcore_axis_name="s")

  @functools.partial(
      pl.kernel,
      out_type=jax.ShapeDtypeStruct((NC, N, D), jnp.float32),
      mesh=mesh,
      scratch_types=[
          pltpu.VMEM((N_CHUNKS, CHUNK), jnp.int32),
          pltpu.VMEM((HALF, CHUNK), jnp.int32),
          pltpu.VMEM((XTRA,), jnp.int32),
          pltpu.VMEM((XTRA,), jnp.int32),
          pltpu.VMEM((CHUNK, D), jnp.float32),
          pltpu.VMEM((CHUNK, D), jnp.float32),
          pltpu.VMEM_SHARED((N, D), jnp.float32),
          pltpu.SemaphoreType.DMA,
          pltpu.SemaphoreType.DMA,
      ],
  )
  def k(h_hbm, src_hbm, dst_hbm, srcx_hbm, dstx_hbm, z_hbm, out_hbm,
        src_v, dst_v, srcx_v, dstx_v, rows0, rows1, acc_sh, sem0, sem1):
    cid = lax.axis_index("c")
    sid = lax.axis_index("s")
    wid = cid * NS + sid
    row0 = sid * STRIPE
    pltpu.sync_copy(z_hbm.at[pl.ds(row0, STRIPE)],
                    acc_sh.at[pl.ds(row0, STRIPE)])

    @pl.when(sid == NS - 1)
    def _():
      pltpu.sync_copy(z_hbm.at[pl.ds(TAIL0, TAIL)], acc_sh.at[pl.ds(TAIL0, TAIL)])

    pltpu.sync_copy(src_hbm.at[wid], src_v)
    pltpu.sync_copy(srcx_hbm.at[wid], srcx_v)
    pltpu.sync_copy(dstx_hbm.at[wid], dstx_v)
    plsc.subcore_barrier()

    bufs = ((rows0, sem0), (rows1, sem1))

    def fire(chunk, buf):
      pltpu.async_copy(h_hbm.at[src_v.at[chunk]], buf[0], buf[1])

    def wait(buf):
      pltpu.make_async_copy(h_hbm.at[pl.ds(0, CHUNK)], buf[0], buf[1]).wait()

    def scatter(buf, c):
      pltpu.sync_copy(buf[0], acc_sh.at[dst_v.at[c]], add=True)

    fire(0, bufs[0])
    for half in range(2):
      pltpu.sync_copy(dst_hbm.at[wid].at[half], dst_v)
      base = half * HALF
      lead, other = bufs[half], bufs[1 - half]

      @pl.loop(0, HALF - 2, step=2)
      def _(c):
        fire(base + c + 1, other)
        wait(lead)
        scatter(lead, c)
        fire(base + c + 2, lead)
        wait(other)
        scatter(other, c + 1)

      if half == 0:
        fire(HALF, other)
      wait(lead)
      scatter(lead, HALF - 1)

    pltpu.async_copy(h_hbm.at[srcx_v], rows0.at[pl.ds(0, XTRA)], sem0).wait()
    pltpu.sync_copy(rows0.at[pl.ds(0, XTRA)], acc_sh.at[dstx_v], add=True)

    plsc.subcore_barrier()
    pltpu.sync_copy(acc_sh.at[pl.ds(row0, STRIPE)],
                    out_hbm.at[cid].at[pl.ds(row0, STRIPE)])

    @pl.when(sid == NS - 1)
    def _():
      pltpu.sync_copy(acc_sh.at[pl.ds(TAIL0, TAIL)],
                      out_hbm.at[cid].at[pl.ds(TAIL0, TAIL)])

  return k(h, src_r, dst_r, srcx, dstx, z)


def _tc_self(h, Wm, Ws, Wu, bias):
  BLK = 1000

  def body(h_ref, wm_ref, ws_ref, wu_ref, b_ref, p_ref, wc_ref):
    p_ref[...] = (jnp.dot(h_ref[...], ws_ref[...],
                          preferred_element_type=jnp.float32) + b_ref[...])
    @pl.when(pl.program_id(0) == 0)
    def _():
      wc_ref[...] = jnp.dot(wm_ref[...], wu_ref[...],
                            preferred_element_type=jnp.float32)

  return pl.pallas_call(
      body,
      grid=(N // BLK,),
      in_specs=[
          pl.BlockSpec((BLK, D), lambda i: (i, 0)),
          pl.BlockSpec((D, D), lambda i: (0, 0)),
          pl.BlockSpec((D, D), lambda i: (0, 0)),
          pl.BlockSpec((D, D), lambda i: (0, 0)),
          pl.BlockSpec((1, D), lambda i: (0, 0)),
      ],
      out_specs=[
          pl.BlockSpec((BLK, D), lambda i: (i, 0)),
          pl.BlockSpec((D, D), lambda i: (0, 0)),
      ],
      out_shape=[
          jax.ShapeDtypeStruct((N, D), jnp.float32),
          jax.ShapeDtypeStruct((D, D), jnp.float32),
      ],
  )(h, Wm, Ws, Wu, bias)


def _tc_combine(P, A, Wc):
  BLK = 1000

  def body(p_ref, a0_ref, a1_ref, wc_ref, o_ref):
    a = a0_ref[...] + a1_ref[...]
    out = p_ref[...] + jnp.dot(a, wc_ref[...],
                               preferred_element_type=jnp.float32)
    o_ref[...] = jnp.maximum(out, 0.0)

  return pl.pallas_call(
      body,
      grid=(N // BLK,),
      in_specs=[
          pl.BlockSpec((BLK, D), lambda i: (i, 0)),
          pl.BlockSpec((BLK, D), lambda i: (i, 0)),
          pl.BlockSpec((BLK, D), lambda i: (i, 0)),
          pl.BlockSpec((D, D), lambda i: (0, 0)),
      ],
      out_specs=pl.BlockSpec((BLK, D), lambda i: (i, 0)),
      out_shape=jax.ShapeDtypeStruct((N, D), jnp.float32),
  )(P, A[0], A[1], Wc)


def kernel(x, edge_index, W_msg, W_self, W_upd, b):
  ei = edge_index.astype(jnp.int32).reshape(2, NW, E_TILE)
  main = N_CHUNKS * CHUNK
  src = ei[0, :, :main].reshape(NW, N_CHUNKS, CHUNK)
  dst = ei[1, :, :main].reshape(NW, 2, HALF, CHUNK)
  srcx = ei[0, :, main:]
  dstx = ei[1, :, main:]
  z = jnp.zeros((N, D), jnp.float32)
  bias = b.reshape(NL, 1, D)
  h = x
  for l in range(NL):
    A = _sc_partial_segsum(h, src, dst, srcx, dstx, z)
    P, Wc = _tc_self(h, W_msg[l], W_self[l], W_upd[l], bias[l])
    h = _tc_combine(P, A, Wc)
  return h


# speedup vs baseline: 4.3533x; 1.0809x over previous
import functools

import jax
import jax.numpy as jnp
from jax import lax
from jax.experimental import pallas as pl
from jax.experimental.pallas import tpu as pltpu
from jax.experimental.pallas import tpu_sc as plsc

N = 10000
E = 320000
D = 128
NL = 3

NC = 2
NS = 16
NW = NC * NS

E_TILE = E // NW
CHUNK = 128
N_CHUNKS = 78
HALF = N_CHUNKS // 2
XTRA = E_TILE - N_CHUNKS * CHUNK
STRIPE = 624
TAIL0 = NS * STRIPE
TAIL = N - TAIL0


def _sc_partial_segsum(h, src_r, dst_r, srcx, dstx, z):
  mesh = plsc.VectorSubcoreMesh(core_axis_name="c", subcore_axis_name="s")

  @functools.partial(
      pl.kernel,
      out_type=jax.ShapeDtypeStruct((NC, N, D), jnp.float32),
      mesh=mesh,
      scratch_types=[
          pltpu.VMEM((N_CHUNKS, CHUNK), jnp.int32),
          pltpu.VMEM((HALF, CHUNK), jnp.int32),
          pltpu.VMEM((XTRA,), jnp.int32),
          pltpu.VMEM((XTRA,), jnp.int32),
          pltpu.VMEM((CHUNK, D), jnp.float32),
          pltpu.VMEM((CHUNK, D), jnp.float32),
          pltpu.VMEM_SHARED((N, D), jnp.float32),
          pltpu.SemaphoreType.DMA,
          pltpu.SemaphoreType.DMA,
      ],
  )
  def k(h_hbm, src_hbm, dst_hbm, srcx_hbm, dstx_hbm, z_hbm, out_hbm,
        src_v, dst_v, srcx_v, dstx_v, rows0, rows1, acc_sh, sem0, sem1):
    cid = lax.axis_index("c")
    sid = lax.axis_index("s")
    wid = cid * NS + sid
    row0 = sid * STRIPE
    pltpu.sync_copy(z_hbm.at[pl.ds(row0, STRIPE)],
                    acc_sh.at[pl.ds(row0, STRIPE)])

    @pl.when(sid == NS - 1)
    def _():
      pltpu.sync_copy(z_hbm.at[pl.ds(TAIL0, TAIL)], acc_sh.at[pl.ds(TAIL0, TAIL)])

    pltpu.sync_copy(src_hbm.at[wid], src_v)
    pltpu.sync_copy(srcx_hbm.at[wid], srcx_v)
    pltpu.sync_copy(dstx_hbm.at[wid], dstx_v)
    plsc.subcore_barrier()

    bufs = ((rows0, sem0), (rows1, sem1))

    def fire(chunk, buf):
      pltpu.async_copy(h_hbm.at[src_v.at[chunk]], buf[0], buf[1])

    def wait(buf):
      pltpu.make_async_copy(h_hbm.at[pl.ds(0, CHUNK)], buf[0], buf[1]).wait()

    def scatter(buf, c):
      pltpu.sync_copy(buf[0], acc_sh.at[dst_v.at[c]], add=True)

    fire(0, bufs[0])
    for half in range(2):
      pltpu.sync_copy(dst_hbm.at[wid].at[half], dst_v)
      base = half * HALF
      lead, other = bufs[half], bufs[1 - half]

      @pl.loop(0, HALF - 2, step=2)
      def _(c):
        fire(base + c + 1, other)
        wait(lead)
        scatter(lead, c)
        fire(base + c + 2, lead)
        wait(other)
        scatter(other, c + 1)

      if half == 0:
        fire(HALF, other)
      wait(lead)
      scatter(lead, HALF - 1)

    pltpu.async_copy(h_hbm.at[srcx_v], rows0.at[pl.ds(0, XTRA)], sem0).wait()
    pltpu.sync_copy(rows0.at[pl.ds(0, XTRA)], acc_sh.at[dstx_v], add=True)

    plsc.subcore_barrier()
    pltpu.sync_copy(acc_sh.at[pl.ds(row0, STRIPE)],
                    out_hbm.at[cid].at[pl.ds(row0, STRIPE)])

    @pl.when(sid == NS - 1)
    def _():
      pltpu.sync_copy(acc_sh.at[pl.ds(TAIL0, TAIL)],
                      out_hbm.at[cid].at[pl.ds(TAIL0, TAIL)])

  return k(h, src_r, dst_r, srcx, dstx, z)


def _tc_self(h, Wm, Ws, Wu, bias):
  BLK = 1000

  def body(h_ref, wm_ref, ws_ref, wu_ref, b_ref, p_ref, wc_ref):
    p_ref[...] = (jnp.dot(h_ref[...], ws_ref[...],
                          preferred_element_type=jnp.float32) + b_ref[...])
    @pl.when(pl.program_id(0) == 0)
    def _():
      wc_ref[...] = jnp.dot(wm_ref[...], wu_ref[...],
                            preferred_element_type=jnp.float32)

  return pl.pallas_call(
      body,
      grid=(N // BLK,),
      in_specs=[
          pl.BlockSpec((BLK, D), lambda i: (i, 0)),
          pl.BlockSpec((D, D), lambda i: (0, 0)),
          pl.BlockSpec((D, D), lambda i: (0, 0)),
          pl.BlockSpec((D, D), lambda i: (0, 0)),
          pl.BlockSpec((1, D), lambda i: (0, 0)),
      ],
      out_specs=[
          pl.BlockSpec((BLK, D), lambda i: (i, 0)),
          pl.BlockSpec((D, D), lambda i: (0, 0)),
      ],
      out_shape=[
          jax.ShapeDtypeStruct((N, D), jnp.float32),
          jax.ShapeDtypeStruct((D, D), jnp.float32),
      ],
  )(h, Wm, Ws, Wu, bias)


def _tc_combine(P, A, Wc):
  BLK = 2000

  def body(p_ref, a_ref, wc_ref, o_ref):
    a = a_ref[0] + a_ref[1]
    out = p_ref[...] + jnp.dot(a, wc_ref[...],
                               preferred_element_type=jnp.float32)
    o_ref[...] = jnp.maximum(out, 0.0)

  return pl.pallas_call(
      body,
      grid=(N // BLK,),
      in_specs=[
          pl.BlockSpec((BLK, D), lambda i: (i, 0)),
          pl.BlockSpec((2, BLK, D), lambda i: (0, i, 0)),
          pl.BlockSpec((D, D), lambda i: (0, 0)),
      ],
      out_specs=pl.BlockSpec((BLK, D), lambda i: (i, 0)),
      out_shape=jax.ShapeDtypeStruct((N, D), jnp.float32),
  )(P, A, Wc)


def kernel(x, edge_index, W_msg, W_self, W_upd, b):
  ei = edge_index.astype(jnp.int32).reshape(2, NW, E_TILE)
  main = N_CHUNKS * CHUNK
  src = ei[0, :, :main].reshape(NW, N_CHUNKS, CHUNK)
  dst = ei[1, :, :main].reshape(NW, 2, HALF, CHUNK)
  srcx = ei[0, :, main:]
  dstx = ei[1, :, main:]
  z = jnp.zeros((N, D), jnp.float32)
  bias = b.reshape(NL, 1, D)
  h = x
  for l in range(NL):
    A = _sc_partial_segsum(h, src, dst, srcx, dstx, z)
    P, Wc = _tc_self(h, W_msg[l], W_self[l], W_upd[l], bias[l])
    h = _tc_combine(P, A, Wc)
  return h


# speedup vs baseline: 4.4769x; 1.0284x over previous
import functools

import jax
import jax.numpy as jnp
from jax import lax
from jax.experimental import pallas as pl
from jax.experimental.pallas import tpu as pltpu
from jax.experimental.pallas import tpu_sc as plsc

N = 10000
E = 320000
D = 128
NL = 3

NC = 2
NS = 16
NW = NC * NS

E_TILE = E // NW
CHUNK = 128
N_CHUNKS = 78
HALF = N_CHUNKS // 2
XTRA = E_TILE - N_CHUNKS * CHUNK
STRIPE = 624
TAIL0 = NS * STRIPE
TAIL = N - TAIL0


def _sc_partial_segsum(h, src_r, dst_r, srcx, dstx):
  mesh = plsc.VectorSubcoreMesh(core_axis_name="c", subcore_axis_name="s")

  @functools.partial(
      pl.kernel,
      out_type=jax.ShapeDtypeStruct((NC, N, D), jnp.float32),
      mesh=mesh,
      scratch_types=[
          pltpu.VMEM((N_CHUNKS, CHUNK), jnp.int32),
          pltpu.VMEM((HALF, CHUNK), jnp.int32),
          pltpu.VMEM((XTRA,), jnp.int32),
          pltpu.VMEM((XTRA,), jnp.int32),
          pltpu.VMEM((CHUNK, D), jnp.float32),
          pltpu.VMEM((CHUNK, D), jnp.float32),
          pltpu.VMEM_SHARED((N, D), jnp.float32),
          pltpu.SemaphoreType.DMA,
          pltpu.SemaphoreType.DMA,
      ],
  )
  def k(h_hbm, src_hbm, dst_hbm, srcx_hbm, dstx_hbm, out_hbm,
        src_v, dst_v, srcx_v, dstx_v, rows0, rows1, acc_sh, sem0, sem1):
    cid = lax.axis_index("c")
    sid = lax.axis_index("s")
    wid = cid * NS + sid
    row0 = sid * STRIPE

    bufs = ((rows0, sem0), (rows1, sem1))

    def fire(chunk, buf):
      pltpu.async_copy(h_hbm.at[src_v.at[chunk]], buf[0], buf[1])

    pltpu.sync_copy(src_hbm.at[wid], src_v)
    pltpu.sync_copy(srcx_hbm.at[wid], srcx_v)
    pltpu.sync_copy(dstx_hbm.at[wid], dstx_v)
    fire(0, bufs[0])

    zv = jnp.zeros((16,), jnp.float32)

    @pl.loop(0, CHUNK)
    def _(r):
      for kk in range(D // 16):
        rows1[r, pl.ds(kk * 16, 16)] = zv

    for j in range(STRIPE // CHUNK):
      pltpu.sync_copy(rows1, acc_sh.at[pl.ds(row0 + j * CHUNK, CHUNK)])
    rem = STRIPE % CHUNK
    pltpu.sync_copy(rows1.at[pl.ds(0, rem)],
                    acc_sh.at[pl.ds(row0 + (STRIPE // CHUNK) * CHUNK, rem)])

    @pl.when(sid == NS - 1)
    def _():
      pltpu.sync_copy(rows1.at[pl.ds(0, TAIL)], acc_sh.at[pl.ds(TAIL0, TAIL)])

    plsc.subcore_barrier()

    def wait(buf):
      pltpu.make_async_copy(h_hbm.at[pl.ds(0, CHUNK)], buf[0], buf[1]).wait()

    def scatter(buf, c):
      pltpu.sync_copy(buf[0], acc_sh.at[dst_v.at[c]], add=True)

    for half in range(2):
      pltpu.sync_copy(dst_hbm.at[wid].at[half], dst_v)
      base = half * HALF
      lead, other = bufs[half], bufs[1 - half]

      @pl.loop(0, HALF - 2, step=2)
      def _(c):
        fire(base + c + 1, other)
        wait(lead)
        scatter(lead, c)
        fire(base + c + 2, lead)
        wait(other)
        scatter(other, c + 1)

      if half == 0:
        fire(HALF, other)
      wait(lead)
      scatter(lead, HALF - 1)

    pltpu.async_copy(h_hbm.at[srcx_v], rows0.at[pl.ds(0, XTRA)], sem0).wait()
    pltpu.sync_copy(rows0.at[pl.ds(0, XTRA)], acc_sh.at[dstx_v], add=True)

    plsc.subcore_barrier()
    pltpu.sync_copy(acc_sh.at[pl.ds(row0, STRIPE)],
                    out_hbm.at[cid].at[pl.ds(row0, STRIPE)])

    @pl.when(sid == NS - 1)
    def _():
      pltpu.sync_copy(acc_sh.at[pl.ds(TAIL0, TAIL)],
                      out_hbm.at[cid].at[pl.ds(TAIL0, TAIL)])

  return k(h, src_r, dst_r, srcx, dstx)


def _tc_self(h, Wm, Ws, Wu, bias):
  BLK = 1000

  def body(h_ref, wm_ref, ws_ref, wu_ref, b_ref, p_ref, wc_ref):
    p_ref[...] = (jnp.dot(h_ref[...], ws_ref[...],
                          preferred_element_type=jnp.float32) + b_ref[...])
    @pl.when(pl.program_id(0) == 0)
    def _():
      wc_ref[...] = jnp.dot(wm_ref[...], wu_ref[...],
                            preferred_element_type=jnp.float32)

  return pl.pallas_call(
      body,
      grid=(N // BLK,),
      in_specs=[
          pl.BlockSpec((BLK, D), lambda i: (i, 0)),
          pl.BlockSpec((D, D), lambda i: (0, 0)),
          pl.BlockSpec((D, D), lambda i: (0, 0)),
          pl.BlockSpec((D, D), lambda i: (0, 0)),
          pl.BlockSpec((1, D), lambda i: (0, 0)),
      ],
      out_specs=[
          pl.BlockSpec((BLK, D), lambda i: (i, 0)),
          pl.BlockSpec((D, D), lambda i: (0, 0)),
      ],
      out_shape=[
          jax.ShapeDtypeStruct((N, D), jnp.float32),
          jax.ShapeDtypeStruct((D, D), jnp.float32),
      ],
  )(h, Wm, Ws, Wu, bias)


def _tc_combine(P, A, Wc):
  BLK = 2000

  def body(p_ref, a_ref, wc_ref, o_ref):
    a = a_ref[0] + a_ref[1]
    out = p_ref[...] + jnp.dot(a, wc_ref[...],
                               preferred_element_type=jnp.float32)
    o_ref[...] = jnp.maximum(out, 0.0)

  return pl.pallas_call(
      body,
      grid=(N // BLK,),
      in_specs=[
          pl.BlockSpec((BLK, D), lambda i: (i, 0)),
          pl.BlockSpec((2, BLK, D), lambda i: (0, i, 0)),
          pl.BlockSpec((D, D), lambda i: (0, 0)),
      ],
      out_specs=pl.BlockSpec((BLK, D), lambda i: (i, 0)),
      out_shape=jax.ShapeDtypeStruct((N, D), jnp.float32),
  )(P, A, Wc)


def kernel(x, edge_index, W_msg, W_self, W_upd, b):
  ei = edge_index.astype(jnp.int32).reshape(2, NW, E_TILE)
  main = N_CHUNKS * CHUNK
  src = ei[0, :, :main].reshape(NW, N_CHUNKS, CHUNK)
  dst = ei[1, :, :main].reshape(NW, 2, HALF, CHUNK)
  srcx = ei[0, :, main:]
  dstx = ei[1, :, main:]
  bias = b.reshape(NL, 1, D)
  h = x
  for l in range(NL):
    A = _sc_partial_segsum(h, src, dst, srcx, dstx)
    P, Wc = _tc_self(h, W_msg[l], W_self[l], W_upd[l], bias[l])
    h = _tc_combine(P, A, Wc)
  return h
